# Initial kernel scaffold; baseline (speedup 1.0000x reference)
import jax
import jax.numpy as jnp
from jax.experimental import pallas as pl


def kernel(world_pos, prev_world_pos, mesh_pos, node_type, edge_index, params):
    raise NotImplementedError("write your pallas kernel here")


# speedup vs baseline: 3.7307x; 3.7307x over previous
import functools

import jax
import jax.numpy as jnp
from jax import lax
from jax.experimental import pallas as pl
from jax.experimental.pallas import tpu as pltpu
from jax.experimental.pallas import tpu_sc as plsc

N = 10000
E = 160000
NUM_TYPES = 9
LATENT = 128
N_BLOCKS = 15
OUT = 3

NC = 2
NS = 16
NW = NC * NS
CHUNK = 128
E_PAD = 163840
NCH = E_PAD // (NW * CHUNK)
EPW = E_PAD // NW
N_ACC = 10112
RPS = N_ACC // NS

_TILE_N = 2000
_TILE_E = 2048

@functools.cache
def _sc_mesh():
    return plsc.VectorSubcoreMesh(core_axis_name="c", subcore_axis_name="s",
                                  num_cores=NC, num_subcores=NS)


def _ln(t, g, be):
    mu = jnp.mean(t, axis=-1, keepdims=True)
    d = t - mu
    var = jnp.mean(d * d, axis=-1, keepdims=True)
    return d * lax.rsqrt(var + 1e-5) * g + be


def _dot(a, b):
    return jnp.dot(a, b, preferred_element_type=jnp.float32)


def _enc_node_body(wp, pwp, nt, w1, b1, w2, b2, g, be, w1a, w1b,
                   x_o, xs_o, xd_o):
    vel = wp[...] - pwp[...]
    oh = (nt[...] == lax.broadcasted_iota(jnp.int32, (_TILE_N, NUM_TYPES), 1))
    nf = jnp.concatenate(
        [vel, oh.astype(jnp.float32), jnp.zeros((_TILE_N, 4), jnp.float32)],
        axis=1)
    h = jax.nn.relu(_dot(nf, w1[...]) + b1[...])
    x = _ln(_dot(h, w2[...]) + b2[...], g[...], be[...])
    x_o[...] = x
    xs_o[...] = _dot(x, w1a[...])
    xd_o[...] = _dot(x, w1b[...])


def _row_spec(tile, width):
    return pl.BlockSpec((tile, width), lambda i: (i, 0))


def _fs(a):
    return pl.BlockSpec(a.shape, lambda i: (0, 0))


def _enc_node_call(wp, pwp, nt, w1p, b1, w2, b2, g, be, w1a, w1b):
    grid = (N // _TILE_N,)
    out = jax.ShapeDtypeStruct((N, LATENT), jnp.float32)
    return pl.pallas_call(
        _enc_node_body,
        grid=grid,
        in_specs=[_row_spec(_TILE_N, 3), _row_spec(_TILE_N, 3),
                  _row_spec(_TILE_N, 1)]
        + [_fs(a) for a in (w1p, b1, w2, b2, g, be, w1a, w1b)],
        out_specs=[_row_spec(_TILE_N, LATENT)] * 3,
        out_shape=[out, out, out],
        compiler_params=pltpu.CompilerParams(
            dimension_semantics=("arbitrary",)),
    )(wp, pwp, nt, w1p, b1, w2, b2, g, be, w1a, w1b)


def _edge_feat(ps, pd):
    d = ps[...] - pd[...]
    rw = d[:, 0:3]
    rm = d[:, 3:6]
    nw = jnp.sqrt(jnp.sum(rw * rw, axis=1, keepdims=True))
    nm = jnp.sqrt(jnp.sum(rm * rm, axis=1, keepdims=True))
    return jnp.concatenate([rw, nw, rm, nm], axis=1)


def _edge0_body(hsrc, ps, pd, ew1, eb1, ew2, eb2, eg, ebe,
                w1c, b1, w2, b2, g, be, out):
    ef = _edge_feat(ps, pd)
    eh = jax.nn.relu(_dot(ef, ew1[...]) + eb1[...])
    e = _ln(_dot(eh, ew2[...]) + eb2[...], eg[...], ebe[...])
    h = jax.nn.relu(hsrc[...] + _dot(e, w1c[...]) + b1[...])
    t = _dot(h, w2[...]) + b2[...]
    out[...] = e + _ln(t, g[...], be[...])


def _edge_body(hsrc, e_in, w1c, b1, w2, b2, g, be, out):
    e = e_in[...]
    h = jax.nn.relu(hsrc[...] + _dot(e, w1c[...]) + b1[...])
    t = _dot(h, w2[...]) + b2[...]
    out[...] = e + _ln(t, g[...], be[...])


def _edge0_call(hsrc, ps, pd, encw, blkw):
    grid = (E_PAD // _TILE_E,)
    return pl.pallas_call(
        _edge0_body,
        grid=grid,
        in_specs=[_row_spec(_TILE_E, LATENT), _row_spec(_TILE_E, 16),
                  _row_spec(_TILE_E, 16)]
        + [_fs(a) for a in (*encw, *blkw)],
        out_specs=_row_spec(_TILE_E, LATENT),
        out_shape=jax.ShapeDtypeStruct((E_PAD, LATENT), jnp.float32),
        compiler_params=pltpu.CompilerParams(
            dimension_semantics=("arbitrary",)),
    )(hsrc, ps, pd, *encw, *blkw)


def _edge_call(hsrc, e, blkw):
    grid = (E_PAD // _TILE_E,)
    return pl.pallas_call(
        _edge_body,
        grid=grid,
        in_specs=[_row_spec(_TILE_E, LATENT), _row_spec(_TILE_E, LATENT)]
        + [_fs(a) for a in blkw],
        out_specs=_row_spec(_TILE_E, LATENT),
        out_shape=jax.ShapeDtypeStruct((E_PAD, LATENT), jnp.float32),
        compiler_params=pltpu.CompilerParams(
            dimension_semantics=("arbitrary",)),
    )(hsrc, e, *blkw)


def _node_body(x_in, a0, a1, v1a, v1b, b1, v2, b2, g, be, w1a, w1b,
               x_o, xs_o, xd_o):
    x = x_in[...]
    a = a0[...] + a1[...]
    h = jax.nn.relu(_dot(x, v1a[...]) + _dot(a, v1b[...]) + b1[...])
    xn = x + _ln(_dot(h, v2[...]) + b2[...], g[...], be[...])
    x_o[...] = xn
    xs_o[...] = _dot(xn, w1a[...])
    xd_o[...] = _dot(xn, w1b[...])


def _node_call(x, a0, a1, nodew, w1a, w1b):
    grid = (N // _TILE_N,)
    out = jax.ShapeDtypeStruct((N, LATENT), jnp.float32)
    return pl.pallas_call(
        _node_body,
        grid=grid,
        in_specs=[_row_spec(_TILE_N, LATENT)] * 3
        + [_fs(a) for a in (*nodew, w1a, w1b)],
        out_specs=[_row_spec(_TILE_N, LATENT)] * 3,
        out_shape=[out, out, out],
        compiler_params=pltpu.CompilerParams(
            dimension_semantics=("arbitrary",)),
    )(x, a0, a1, *nodew, w1a, w1b)


def _node_last_body(x_in, a0, a1, v1a, v1b, b1, v2, b2, g, be,
                    d1, db1, d2, db2, wp, pwp, nt, out):
    x = x_in[...]
    a = a0[...] + a1[...]
    h = jax.nn.relu(_dot(x, v1a[...]) + _dot(a, v1b[...]) + b1[...])
    xn = x + _ln(_dot(h, v2[...]) + b2[...], g[...], be[...])
    dh = jax.nn.relu(_dot(xn, d1[...]) + db1[...])
    pred = _dot(dh, d2[...]) + db2[...]
    w = wp[...]
    new = w + (w - pwp[...]) + pred
    out[...] = jnp.where(nt[...] == 0, new, w)


def _node_last_call(x, a0, a1, nodew, decw, wp, pwp, nt):
    grid = (N // _TILE_N,)
    return pl.pallas_call(
        _node_last_body,
        grid=grid,
        in_specs=[_row_spec(_TILE_N, LATENT)] * 3
        + [_fs(a) for a in (*nodew, *decw)]
        + [_row_spec(_TILE_N, 3), _row_spec(_TILE_N, 3),
           _row_spec(_TILE_N, 1)],
        out_specs=_row_spec(_TILE_N, 3),
        out_shape=jax.ShapeDtypeStruct((N, 3), jnp.float32),
        compiler_params=pltpu.CompilerParams(
            dimension_semantics=("arbitrary",)),
    )(x, a0, a1, *nodew, *decw, wp, pwp, nt)


def _wid():
    return lax.axis_index("s") * NC + lax.axis_index("c")


def _sc_pos_gather(table, src3, dst3):
    @functools.partial(
        pl.kernel,
        out_type=(jax.ShapeDtypeStruct((E_PAD, 16), jnp.float32),
                  jax.ShapeDtypeStruct((E_PAD, 16), jnp.float32)),
        mesh=_sc_mesh(),
        scratch_types=(
            pltpu.VMEM((NCH, CHUNK), jnp.int32),
            pltpu.VMEM((NCH, CHUNK), jnp.int32),
            pltpu.VMEM((CHUNK, 16), jnp.float32),
            pltpu.VMEM((CHUNK, 16), jnp.float32),
            pltpu.SemaphoreType.DMA,
            pltpu.SemaphoreType.DMA,
        ),
        compiler_params=pltpu.CompilerParams(use_tc_tiling_on_sc=False),
    )
    def k(tab_h, src_h, dst_h, ps_o, pd_o, idxs, idxd, bufs, bufd, sa, sb):
        w = _wid()
        pltpu.sync_copy(src_h.at[w], idxs)
        pltpu.sync_copy(dst_h.at[w], idxd)

        def step(j, carry):
            a = pltpu.async_copy(tab_h.at[idxs.at[j]], bufs, sa)
            b = pltpu.async_copy(tab_h.at[idxd.at[j]], bufd, sb)
            a.wait()
            b.wait()
            base = w * EPW + j * CHUNK
            pltpu.sync_copy(bufs, ps_o.at[pl.ds(base, CHUNK)])
            pltpu.sync_copy(bufd, pd_o.at[pl.ds(base, CHUNK)])
            return carry

        lax.fori_loop(0, NCH, step, 0)

    return k(table, src3, dst3)


def _sc_gather_add(xs, xd, src3, dst3):
    @functools.partial(
        pl.kernel,
        out_type=jax.ShapeDtypeStruct((E_PAD, LATENT), jnp.float32),
        mesh=_sc_mesh(),
        scratch_types=(
            pltpu.VMEM((NCH, CHUNK), jnp.int32),
            pltpu.VMEM((NCH, CHUNK), jnp.int32),
            pltpu.VMEM((CHUNK, LATENT), jnp.float32),
            pltpu.VMEM((CHUNK, LATENT), jnp.float32),
            pltpu.SemaphoreType.DMA,
            pltpu.SemaphoreType.DMA,
        ),
    )
    def k(xs_h, xd_h, src_h, dst_h, out_h, idxs, idxd, bufa, bufb, sa, sb):
        w = _wid()
        pltpu.sync_copy(src_h.at[w], idxs)
        pltpu.sync_copy(dst_h.at[w], idxd)

        def step(j, carry):
            a = pltpu.async_copy(xs_h.at[idxs.at[j]], bufa, sa)
            b = pltpu.async_copy(xd_h.at[idxd.at[j]], bufb, sb)
            a.wait()
            b.wait()

            def add_row(r, c2):
                for cc in range(LATENT // 16):
                    sl = pl.ds(cc * 16, 16)
                    plsc.addupdate(bufa.at[r, sl], bufb[r, sl])
                return c2

            lax.fori_loop(0, CHUNK, add_row, 0)
            base = w * EPW + j * CHUNK
            pltpu.sync_copy(bufa, out_h.at[pl.ds(base, CHUNK)])
            return carry

        lax.fori_loop(0, NCH, step, 0)

    return k(xs, xd, src3, dst3)


def _sc_segment_sum(e, dst3):
    @functools.partial(
        pl.kernel,
        out_type=(jax.ShapeDtypeStruct((N_ACC, LATENT), jnp.float32),
                  jax.ShapeDtypeStruct((N_ACC, LATENT), jnp.float32)),
        mesh=_sc_mesh(),
        scratch_types=(
            pltpu.VMEM((NCH, CHUNK), jnp.int32),
            pltpu.VMEM((CHUNK, LATENT), jnp.float32),
            pltpu.VMEM_SHARED((N_ACC, LATENT), jnp.float32),
        ),
    )
    def k(e_h, dst_h, a0_o, a1_o, idx, buf, acc):
        c = lax.axis_index("c")
        s = lax.axis_index("s")
        w = s * NC + c

        def zero_row(r, carry):
            for cc in range(LATENT // 16):
                buf[r, pl.ds(cc * 16, 16)] = jnp.zeros((16,), jnp.float32)
            return carry

        lax.fori_loop(0, CHUNK, zero_row, 0)
        base = s * RPS
        for off in range(0, RPS, CHUNK):
            sz = min(CHUNK, RPS - off)
            pltpu.sync_copy(buf.at[pl.ds(0, sz)],
                            acc.at[pl.ds(base + off, sz)])
        plsc.subcore_barrier()

        pltpu.sync_copy(dst_h.at[w], idx)

        def step(j, carry):
            pltpu.sync_copy(e_h.at[pl.ds(w * EPW + j * CHUNK, CHUNK)], buf)
            pltpu.sync_copy(buf, acc.at[idx.at[j]], add=True)
            return carry

        lax.fori_loop(0, NCH, step, 0)
        plsc.subcore_barrier()

        @pl.when(c == 0)
        def _():
            pltpu.sync_copy(acc.at[pl.ds(base, RPS)],
                            a0_o.at[pl.ds(base, RPS)])

        @pl.when(c == 1)
        def _():
            pltpu.sync_copy(acc.at[pl.ds(base, RPS)],
                            a1_o.at[pl.ds(base, RPS)])

    return k(e, dst3)


def _mlp_w(p):
    return (p["W1"], p["b1"].reshape(1, -1), p["W2"], p["b2"].reshape(1, -1),
            p["g"].reshape(1, -1), p["be"].reshape(1, -1))


def kernel(world_pos, prev_world_pos, mesh_pos, node_type, edge_index, params):
    f32 = jnp.float32
    src = edge_index[0].astype(jnp.int32)
    dst = edge_index[1].astype(jnp.int32)
    pad = E_PAD - E
    spread = (jnp.arange(pad, dtype=jnp.int32) * 131) % N
    src3 = jnp.concatenate([src, spread]).reshape(NW, NCH, CHUNK)
    dstg3 = jnp.concatenate([dst, spread]).reshape(NW, NCH, CHUNK)
    dsts3 = jnp.concatenate(
        [dst, N + (jnp.arange(pad, dtype=jnp.int32) % (N_ACC - N))]
    ).reshape(NW, NCH, CHUNK)

    wp = world_pos.astype(f32)
    pwp = prev_world_pos.astype(f32)
    mp = mesh_pos.astype(f32)
    nt = node_type.astype(jnp.int32).reshape(N, 1)
    p16 = jnp.concatenate([wp, mp, jnp.zeros((N, 10), f32)], axis=1)

    pn = params["enc_node"]
    w1p = jnp.concatenate([pn["W1"], jnp.zeros((4, LATENT), f32)], axis=0)
    encn = (w1p, pn["b1"].reshape(1, -1), pn["W2"], pn["b2"].reshape(1, -1),
            pn["g"].reshape(1, -1), pn["be"].reshape(1, -1))
    ence = _mlp_w(params["enc_edge"])

    edge_w = []
    for pe in params["blocks_edge"]:
        w1 = pe["W1"]
        edge_w.append(dict(
            w1a=w1[0:LATENT], w1b=w1[LATENT:2 * LATENT],
            blk=(w1[2 * LATENT:3 * LATENT], pe["b1"].reshape(1, -1),
                 pe["W2"], pe["b2"].reshape(1, -1), pe["g"].reshape(1, -1),
                 pe["be"].reshape(1, -1))))
    node_w = []
    for pnb in params["blocks_node"]:
        v1 = pnb["W1"]
        node_w.append((v1[0:LATENT], v1[LATENT:2 * LATENT],
                       pnb["b1"].reshape(1, -1), pnb["W2"],
                       pnb["b2"].reshape(1, -1), pnb["g"].reshape(1, -1),
                       pnb["be"].reshape(1, -1)))
    d = params["dec"]
    decw = (d["W1"], d["b1"].reshape(1, -1), d["W2"], d["b2"].reshape(1, -1))

    x, xs, xd = _enc_node_call(wp, pwp, nt, *encn,
                               edge_w[0]["w1a"], edge_w[0]["w1b"])
    psrc, pdst = _sc_pos_gather(p16, src3, dstg3)

    e = None
    for b in range(N_BLOCKS):
        hsrc = _sc_gather_add(xs, xd, src3, dstg3)
        if b == 0:
            e = _edge0_call(hsrc, psrc, pdst, ence, edge_w[0]["blk"])
        else:
            e = _edge_call(hsrc, e, edge_w[b]["blk"])
        a0, a1 = _sc_segment_sum(e, dsts3)
        if b < N_BLOCKS - 1:
            x, xs, xd = _node_call(x, a0, a1, node_w[b],
                                   edge_w[b + 1]["w1a"], edge_w[b + 1]["w1b"])
        else:
            out = _node_last_call(x, a0, a1, node_w[b], decw, wp, pwp, nt)
    return out


# speedup vs baseline: 5.0477x; 1.3530x over previous
import functools

import jax
import jax.numpy as jnp
from jax import lax
from jax.experimental import pallas as pl
from jax.experimental.pallas import tpu as pltpu
from jax.experimental.pallas import tpu_sc as plsc

N = 10000
E = 160000
NUM_TYPES = 9
LATENT = 128
N_BLOCKS = 15
OUT = 3

NC = 2
NS = 16
NW = NC * NS
CHUNK = 128
E_PAD = 163840
NCH = E_PAD // (NW * CHUNK)
EPW = E_PAD // NW
N_ACC = 10112
RPS = N_ACC // NS

EH = E_PAD // 2
NCHH = NCH // 2
EPWH = EH // NW

_TILE_N = 2000
_TILE_E = 4096

@functools.cache
def _sc_mesh():
    return plsc.VectorSubcoreMesh(core_axis_name="c", subcore_axis_name="s",
                                  num_cores=NC, num_subcores=NS)


def _ln(t, g, be):
    mu = jnp.mean(t, axis=-1, keepdims=True)
    d = t - mu
    var = jnp.mean(d * d, axis=-1, keepdims=True)
    return d * lax.rsqrt(var + 1e-5) * g + be


def _dot(a, b):
    return jnp.dot(a, b, preferred_element_type=jnp.float32)


def _enc_node_body(wp, pwp, nt, w1, b1, w2, b2, g, be, w1a, w1b,
                   x_o, xs_o, xd_o):
    vel = wp[...] - pwp[...]
    oh = (nt[...] == lax.broadcasted_iota(jnp.int32, (_TILE_N, NUM_TYPES), 1))
    nf = jnp.concatenate(
        [vel, oh.astype(jnp.float32), jnp.zeros((_TILE_N, 4), jnp.float32)],
        axis=1)
    h = jax.nn.relu(_dot(nf, w1[...]) + b1[...])
    x = _ln(_dot(h, w2[...]) + b2[...], g[...], be[...])
    x_o[...] = x
    xs_o[...] = _dot(x, w1a[...])
    xd_o[...] = _dot(x, w1b[...])


def _row_spec(tile, width):
    return pl.BlockSpec((tile, width), lambda i: (i, 0))


def _fs(a):
    return pl.BlockSpec(a.shape, lambda i: (0, 0))


def _enc_node_call(wp, pwp, nt, w1p, b1, w2, b2, g, be, w1a, w1b):
    grid = (N // _TILE_N,)
    out = jax.ShapeDtypeStruct((N, LATENT), jnp.float32)
    return pl.pallas_call(
        _enc_node_body,
        grid=grid,
        in_specs=[_row_spec(_TILE_N, 3), _row_spec(_TILE_N, 3),
                  _row_spec(_TILE_N, 1)]
        + [_fs(a) for a in (w1p, b1, w2, b2, g, be, w1a, w1b)],
        out_specs=[_row_spec(_TILE_N, LATENT)] * 3,
        out_shape=[out, out, out],
        compiler_params=pltpu.CompilerParams(
            dimension_semantics=("arbitrary",)),
    )(wp, pwp, nt, w1p, b1, w2, b2, g, be, w1a, w1b)


def _edge_feat(ps, pd):
    d = ps[...] - pd[...]
    rw = d[:, 0:3]
    rm = d[:, 3:6]
    nw = jnp.sqrt(jnp.sum(rw * rw, axis=1, keepdims=True))
    nm = jnp.sqrt(jnp.sum(rm * rm, axis=1, keepdims=True))
    return jnp.concatenate([rw, nw, rm, nm], axis=1)


def _edge0_body(hsrc, ps, pd, ew1, eb1, ew2, eb2, eg, ebe,
                w1c, b1, w2, b2, g, be, out):
    ef = _edge_feat(ps, pd)
    eh = jax.nn.relu(_dot(ef, ew1[...]) + eb1[...])
    e = _ln(_dot(eh, ew2[...]) + eb2[...], eg[...], ebe[...])
    h = jax.nn.relu(hsrc[...] + _dot(e, w1c[...]) + b1[...])
    t = _dot(h, w2[...]) + b2[...]
    out[...] = e + _ln(t, g[...], be[...])


def _edge_body(hsrc, e_in, w1c, b1, w2, b2, g, be, out):
    e = e_in[...]
    h = jax.nn.relu(hsrc[...] + _dot(e, w1c[...]) + b1[...])
    t = _dot(h, w2[...]) + b2[...]
    out[...] = e + _ln(t, g[...], be[...])


def _edge0_call(hsrc, ps, pd, encw, blkw):
    grid = (EH // _TILE_E,)
    return pl.pallas_call(
        _edge0_body,
        grid=grid,
        in_specs=[_row_spec(_TILE_E, LATENT), _row_spec(_TILE_E, 16),
                  _row_spec(_TILE_E, 16)]
        + [_fs(a) for a in (*encw, *blkw)],
        out_specs=_row_spec(_TILE_E, LATENT),
        out_shape=jax.ShapeDtypeStruct((EH, LATENT), jnp.float32),
        compiler_params=pltpu.CompilerParams(
            dimension_semantics=("arbitrary",)),
    )(hsrc, ps, pd, *encw, *blkw)


def _edge_call(hsrc, e, blkw):
    grid = (EH // _TILE_E,)
    return pl.pallas_call(
        _edge_body,
        grid=grid,
        in_specs=[_row_spec(_TILE_E, LATENT), _row_spec(_TILE_E, LATENT)]
        + [_fs(a) for a in blkw],
        out_specs=_row_spec(_TILE_E, LATENT),
        out_shape=jax.ShapeDtypeStruct((EH, LATENT), jnp.float32),
        compiler_params=pltpu.CompilerParams(
            dimension_semantics=("arbitrary",)),
    )(hsrc, e, *blkw)


def _node_body(x_in, a0, a1, a2, a3, v1a, v1b, b1, v2, b2, g, be, w1a, w1b,
               x_o, xs_o, xd_o):
    x = x_in[...]
    a = (a0[...] + a1[...]) + (a2[...] + a3[...])
    h = jax.nn.relu(_dot(x, v1a[...]) + _dot(a, v1b[...]) + b1[...])
    xn = x + _ln(_dot(h, v2[...]) + b2[...], g[...], be[...])
    x_o[...] = xn
    xs_o[...] = _dot(xn, w1a[...])
    xd_o[...] = _dot(xn, w1b[...])


def _node_call(x, aggs, nodew, w1a, w1b):
    grid = (N // _TILE_N,)
    out = jax.ShapeDtypeStruct((N, LATENT), jnp.float32)
    return pl.pallas_call(
        _node_body,
        grid=grid,
        in_specs=[_row_spec(_TILE_N, LATENT)] * 5
        + [_fs(a) for a in (*nodew, w1a, w1b)],
        out_specs=[_row_spec(_TILE_N, LATENT)] * 3,
        out_shape=[out, out, out],
        compiler_params=pltpu.CompilerParams(
            dimension_semantics=("arbitrary",)),
    )(x, *aggs, *nodew, w1a, w1b)


def _node_last_body(x_in, a0, a1, a2, a3, v1a, v1b, b1, v2, b2, g, be,
                    d1, db1, d2, db2, wp, pwp, nt, out):
    x = x_in[...]
    a = (a0[...] + a1[...]) + (a2[...] + a3[...])
    h = jax.nn.relu(_dot(x, v1a[...]) + _dot(a, v1b[...]) + b1[...])
    xn = x + _ln(_dot(h, v2[...]) + b2[...], g[...], be[...])
    dh = jax.nn.relu(_dot(xn, d1[...]) + db1[...])
    pred = _dot(dh, d2[...]) + db2[...]
    w = wp[...]
    new = w + (w - pwp[...]) + pred
    out[...] = jnp.where(nt[...] == 0, new, w)


def _node_last_call(x, aggs, nodew, decw, wp, pwp, nt):
    grid = (N // _TILE_N,)
    return pl.pallas_call(
        _node_last_body,
        grid=grid,
        in_specs=[_row_spec(_TILE_N, LATENT)] * 5
        + [_fs(a) for a in (*nodew, *decw)]
        + [_row_spec(_TILE_N, 3), _row_spec(_TILE_N, 3),
           _row_spec(_TILE_N, 1)],
        out_specs=_row_spec(_TILE_N, 3),
        out_shape=jax.ShapeDtypeStruct((N, 3), jnp.float32),
        compiler_params=pltpu.CompilerParams(
            dimension_semantics=("arbitrary",)),
    )(x, *aggs, *nodew, *decw, wp, pwp, nt)


def _wid():
    return lax.axis_index("s") * NC + lax.axis_index("c")


def _sc_pos_gather(table, src3, dst3):
    @functools.partial(
        pl.kernel,
        out_type=(jax.ShapeDtypeStruct((EH, 16), jnp.float32),
                  jax.ShapeDtypeStruct((EH, 16), jnp.float32)),
        mesh=_sc_mesh(),
        scratch_types=(
            pltpu.VMEM((NCHH, CHUNK), jnp.int32),
            pltpu.VMEM((NCHH, CHUNK), jnp.int32),
            pltpu.VMEM((2, CHUNK, 16), jnp.float32),
            pltpu.VMEM((2, CHUNK, 16), jnp.float32),
            (pltpu.SemaphoreType.DMA, pltpu.SemaphoreType.DMA),
            (pltpu.SemaphoreType.DMA, pltpu.SemaphoreType.DMA),
            (pltpu.SemaphoreType.DMA, pltpu.SemaphoreType.DMA),
            (pltpu.SemaphoreType.DMA, pltpu.SemaphoreType.DMA),
        ),
        compiler_params=pltpu.CompilerParams(use_tc_tiling_on_sc=False),
    )
    def k(tab_h, src_h, dst_h, ps_o, pd_o, idxs, idxd, bufs, bufd,
          sga, sgb, soa, sob):
        w = _wid()
        pltpu.sync_copy(src_h.at[w], idxs)
        pltpu.sync_copy(dst_h.at[w], idxd)

        def issue(j, sl):
            return (pltpu.async_copy(tab_h.at[idxs.at[j]], bufs.at[sl],
                                     sga[sl]),
                    pltpu.async_copy(tab_h.at[idxd.at[j]], bufd.at[sl],
                                     sgb[sl]))

        gd = {0: issue(0, 0)}
        od = {}
        for j in range(NCHH):
            sl = j % 2
            if j >= 1:
                for d in od[j - 1]:
                    d.wait()
            if j + 1 < NCHH:
                gd[j + 1] = issue(j + 1, (j + 1) % 2)
            for d in gd[j]:
                d.wait()
            base = w * EPWH + j * CHUNK
            od[j] = (
                pltpu.async_copy(bufs.at[sl], ps_o.at[pl.ds(base, CHUNK)],
                                 soa[sl]),
                pltpu.async_copy(bufd.at[sl], pd_o.at[pl.ds(base, CHUNK)],
                                 sob[sl]),
            )
        for d in od[NCHH - 1]:
            d.wait()

    return k(table, src3, dst3)


def _sc_gather_add(xs, xd, src3, dst3):
    NP = NCHH // 2

    @functools.partial(
        pl.kernel,
        out_type=jax.ShapeDtypeStruct((EH, LATENT), jnp.float32),
        mesh=_sc_mesh(),
        scratch_types=(
            pltpu.VMEM((NCHH, CHUNK), jnp.int32),
            pltpu.VMEM((NCHH, CHUNK), jnp.int32),
            pltpu.VMEM((2, 2 * CHUNK, LATENT), jnp.float32),
            pltpu.VMEM((2, CHUNK, LATENT), jnp.float32),
            (pltpu.SemaphoreType.DMA, pltpu.SemaphoreType.DMA),
            (pltpu.SemaphoreType.DMA, pltpu.SemaphoreType.DMA),
            (pltpu.SemaphoreType.DMA, pltpu.SemaphoreType.DMA),
        ),
    )
    def k(xs_h, xd_h, src_h, dst_h, out_h, idxs, idxd, bufa, bufb,
          sga, sgb, so):
        w = _wid()
        pltpu.sync_copy(src_h.at[w], idxs)
        pltpu.sync_copy(dst_h.at[w], idxd)

        def issue(j):
            ss, hh = (j // 2) % 2, (j % 2) * CHUNK
            return (
                pltpu.async_copy(xs_h.at[idxs.at[j]],
                                 bufa.at[ss, pl.ds(hh, CHUNK)], sga[j % 2]),
                pltpu.async_copy(xd_h.at[idxd.at[j]], bufb.at[j % 2],
                                 sgb[j % 2]),
            )

        gd = {0: issue(0), 1: issue(1)}
        od = {}
        for p in range(NP):
            ss = p % 2
            if p >= 1:
                od[p - 1].wait()
            for j in (2 * p, 2 * p + 1):
                for d in gd[j]:
                    d.wait()

            for half in range(2):
                jc = 2 * p + half

                def add_row(r4, carry, _ss=ss, _h=half * CHUNK, _b=jc % 2):
                    for rr in range(4):
                        r = r4 * 4 + rr
                        for cc in range(LATENT // 16):
                            cs = pl.ds(cc * 16, 16)
                            plsc.addupdate(bufa.at[_ss, _h + r, cs],
                                           bufb[_b, r, cs])
                    return carry

                lax.fori_loop(0, CHUNK // 4, add_row, 0)

            for j in (2 * p + 2, 2 * p + 3):
                if j < NCHH:
                    gd[j] = issue(j)
            base = w * EPWH + p * 2 * CHUNK
            od[p] = pltpu.async_copy(bufa.at[ss],
                                     out_h.at[pl.ds(base, 2 * CHUNK)],
                                     so[ss])
        od[NP - 1].wait()

    return k(xs, xd, src3, dst3)


def _sc_segment_sum(e, dst3):
    @functools.partial(
        pl.kernel,
        out_type=(jax.ShapeDtypeStruct((N_ACC, LATENT), jnp.float32),
                  jax.ShapeDtypeStruct((N_ACC, LATENT), jnp.float32)),
        mesh=_sc_mesh(),
        scratch_types=(
            pltpu.VMEM((NCHH, CHUNK), jnp.int32),
            pltpu.VMEM((2, CHUNK, LATENT), jnp.float32),
            pltpu.VMEM_SHARED((N_ACC, LATENT), jnp.float32),
            tuple(pltpu.SemaphoreType.DMA for _ in range(2)),
            tuple(pltpu.SemaphoreType.DMA for _ in range(2)),
        ),
    )
    def k(e_h, dst_h, a0_o, a1_o, idx, buf, acc, sld, ssc):
        c = lax.axis_index("c")
        s = lax.axis_index("s")
        w = s * NC + c

        def zero_row(r, carry):
            for cc in range(LATENT // 16):
                buf[0, r, pl.ds(cc * 16, 16)] = jnp.zeros((16,), jnp.float32)
            return carry

        lax.fori_loop(0, CHUNK, zero_row, 0)
        base = s * RPS
        for off in range(0, RPS, CHUNK):
            sz = min(CHUNK, RPS - off)
            pltpu.sync_copy(buf.at[0, pl.ds(0, sz)],
                            acc.at[pl.ds(base + off, sz)])
        plsc.subcore_barrier()

        pltpu.sync_copy(dst_h.at[w], idx)

        def load(j, sl):
            return pltpu.async_copy(
                e_h.at[pl.ds(w * EPWH + j * CHUNK, CHUNK)], buf.at[sl],
                sld[sl])

        ld = {0: load(0, 0)}
        sd = {}
        for j in range(NCHH):
            sl = j % 2
            if j >= 1:
                sd[j - 1].wait()
            if j + 1 < NCHH:
                ld[j + 1] = load(j + 1, (j + 1) % 2)
            ld[j].wait()
            sd[j] = pltpu.async_copy(buf.at[sl], acc.at[idx.at[j]], ssc[sl],
                                     add=True)
        sd[NCHH - 1].wait()
        plsc.subcore_barrier()

        @pl.when(c == 0)
        def _():
            pltpu.sync_copy(acc.at[pl.ds(base, RPS)],
                            a0_o.at[pl.ds(base, RPS)])

        @pl.when(c == 1)
        def _():
            pltpu.sync_copy(acc.at[pl.ds(base, RPS)],
                            a1_o.at[pl.ds(base, RPS)])

    return k(e, dst3)


def _mlp_w(p):
    return (p["W1"], p["b1"].reshape(1, -1), p["W2"], p["b2"].reshape(1, -1),
            p["g"].reshape(1, -1), p["be"].reshape(1, -1))


def kernel(world_pos, prev_world_pos, mesh_pos, node_type, edge_index, params):
    f32 = jnp.float32
    src = edge_index[0].astype(jnp.int32)
    dst = edge_index[1].astype(jnp.int32)
    pad = E_PAD - E
    spread = (jnp.arange(pad, dtype=jnp.int32) * 131) % N
    src4 = jnp.concatenate([src, spread]).reshape(2, NW, NCHH, CHUNK)
    dstg4 = jnp.concatenate([dst, spread]).reshape(2, NW, NCHH, CHUNK)
    dsts4 = jnp.concatenate(
        [dst, N + (jnp.arange(pad, dtype=jnp.int32) % (N_ACC - N))]
    ).reshape(2, NW, NCHH, CHUNK)

    wp = world_pos.astype(f32)
    pwp = prev_world_pos.astype(f32)
    mp = mesh_pos.astype(f32)
    nt = node_type.astype(jnp.int32).reshape(N, 1)
    p16 = jnp.concatenate([wp, mp, jnp.zeros((N, 10), f32)], axis=1)

    pn = params["enc_node"]
    w1p = jnp.concatenate([pn["W1"], jnp.zeros((4, LATENT), f32)], axis=0)
    encn = (w1p, pn["b1"].reshape(1, -1), pn["W2"], pn["b2"].reshape(1, -1),
            pn["g"].reshape(1, -1), pn["be"].reshape(1, -1))
    ence = _mlp_w(params["enc_edge"])

    edge_w = []
    for pe in params["blocks_edge"]:
        w1 = pe["W1"]
        edge_w.append(dict(
            w1a=w1[0:LATENT], w1b=w1[LATENT:2 * LATENT],
            blk=(w1[2 * LATENT:3 * LATENT], pe["b1"].reshape(1, -1),
                 pe["W2"], pe["b2"].reshape(1, -1), pe["g"].reshape(1, -1),
                 pe["be"].reshape(1, -1))))
    node_w = []
    for pnb in params["blocks_node"]:
        v1 = pnb["W1"]
        node_w.append((v1[0:LATENT], v1[LATENT:2 * LATENT],
                       pnb["b1"].reshape(1, -1), pnb["W2"],
                       pnb["b2"].reshape(1, -1), pnb["g"].reshape(1, -1),
                       pnb["be"].reshape(1, -1)))
    d = params["dec"]
    decw = (d["W1"], d["b1"].reshape(1, -1), d["W2"], d["b2"].reshape(1, -1))

    x, xs, xd = _enc_node_call(wp, pwp, nt, *encn,
                               edge_w[0]["w1a"], edge_w[0]["w1b"])
    pos = [_sc_pos_gather(p16, src4[h], dstg4[h]) for h in range(2)]

    eh = [None, None]
    for b in range(N_BLOCKS):
        aggs = []
        for h in range(2):
            hsrc = _sc_gather_add(xs, xd, src4[h], dstg4[h])
            if b == 0:
                eh[h] = _edge0_call(hsrc, pos[h][0], pos[h][1], ence,
                                    edge_w[0]["blk"])
            else:
                eh[h] = _edge_call(hsrc, eh[h], edge_w[b]["blk"])
            aggs.extend(_sc_segment_sum(eh[h], dsts4[h]))
        if b < N_BLOCKS - 1:
            x, xs, xd = _node_call(x, aggs, node_w[b],
                                   edge_w[b + 1]["w1a"], edge_w[b + 1]["w1b"])
        else:
            out = _node_last_call(x, aggs, node_w[b], decw, wp, pwp, nt)
    return out


# speedup vs baseline: 5.4813x; 1.0859x over previous
import functools

import jax
import jax.numpy as jnp
from jax import lax
from jax.experimental import pallas as pl
from jax.experimental.pallas import tpu as pltpu
from jax.experimental.pallas import tpu_sc as plsc

N = 10000
E = 160000
NUM_TYPES = 9
LATENT = 128
N_BLOCKS = 15
OUT = 3

NC = 2
NS = 16
NW = NC * NS
CHUNK = 128
E_PAD = 163840
NCH = E_PAD // (NW * CHUNK)
EPW = E_PAD // NW
N_ACC = 10112
RPS = N_ACC // NS

EH = E_PAD // 2
NCHH = NCH // 2
EPWH = EH // NW

_TILE_N = 2000
_TILE_E = 4096

@functools.cache
def _sc_mesh():
    return plsc.VectorSubcoreMesh(core_axis_name="c", subcore_axis_name="s",
                                  num_cores=NC, num_subcores=NS)


def _ln(t, g, be):
    mu = jnp.mean(t, axis=-1, keepdims=True)
    d = t - mu
    var = jnp.mean(d * d, axis=-1, keepdims=True)
    return d * lax.rsqrt(var + 1e-5) * g + be


def _dot(a, b):
    return jnp.dot(a, b, preferred_element_type=jnp.float32)


def _enc_node_body(wp, pwp, nt, w1, b1, w2, b2, g, be, w1a, w1b,
                   x_o, xs_o, xd_o):
    vel = wp[...] - pwp[...]
    oh = (nt[...] == lax.broadcasted_iota(jnp.int32, (_TILE_N, NUM_TYPES), 1))
    nf = jnp.concatenate(
        [vel, oh.astype(jnp.float32), jnp.zeros((_TILE_N, 4), jnp.float32)],
        axis=1)
    h = jax.nn.relu(_dot(nf, w1[...]) + b1[...])
    x = _ln(_dot(h, w2[...]) + b2[...], g[...], be[...])
    x_o[...] = x
    xs_o[...] = _dot(x, w1a[...])
    xd_o[...] = _dot(x, w1b[...])


def _row_spec(tile, width):
    return pl.BlockSpec((tile, width), lambda i: (i, 0))


def _fs(a):
    return pl.BlockSpec(a.shape, lambda i: (0, 0))


def _enc_node_call(wp, pwp, nt, w1p, b1, w2, b2, g, be, w1a, w1b):
    grid = (N // _TILE_N,)
    out = jax.ShapeDtypeStruct((N, LATENT), jnp.float32)
    return pl.pallas_call(
        _enc_node_body,
        grid=grid,
        in_specs=[_row_spec(_TILE_N, 3), _row_spec(_TILE_N, 3),
                  _row_spec(_TILE_N, 1)]
        + [_fs(a) for a in (w1p, b1, w2, b2, g, be, w1a, w1b)],
        out_specs=[_row_spec(_TILE_N, LATENT)] * 3,
        out_shape=[out, out, out],
        compiler_params=pltpu.CompilerParams(
            dimension_semantics=("arbitrary",)),
    )(wp, pwp, nt, w1p, b1, w2, b2, g, be, w1a, w1b)


def _edge_feat(ps, pd):
    d = ps[...] - pd[...]
    rw = d[:, 0:3]
    rm = d[:, 3:6]
    nw = jnp.sqrt(jnp.sum(rw * rw, axis=1, keepdims=True))
    nm = jnp.sqrt(jnp.sum(rm * rm, axis=1, keepdims=True))
    return jnp.concatenate([rw, nw, rm, nm], axis=1)


def _edge0_body(hsrc, ps, pd, ew1, eb1, ew2, eb2, eg, ebe,
                w1c, b1, w2, b2, g, be, out):
    ef = _edge_feat(ps, pd)
    eh = jax.nn.relu(_dot(ef, ew1[...]) + eb1[...])
    e = _ln(_dot(eh, ew2[...]) + eb2[...], eg[...], ebe[...])
    h = jax.nn.relu(hsrc[...] + _dot(e, w1c[...]) + b1[...])
    t = _dot(h, w2[...]) + b2[...]
    out[...] = e + _ln(t, g[...], be[...])


def _edge_body(hsrc, e_in, w1c, b1, w2, b2, g, be, out):
    e = e_in[...]
    h = jax.nn.relu(hsrc[...] + _dot(e, w1c[...]) + b1[...])
    t = _dot(h, w2[...]) + b2[...]
    out[...] = e + _ln(t, g[...], be[...])


def _edge0_call(hsrc, ps, pd, encw, blkw):
    grid = (EH // _TILE_E,)
    return pl.pallas_call(
        _edge0_body,
        grid=grid,
        in_specs=[_row_spec(_TILE_E, LATENT), _row_spec(_TILE_E, 16),
                  _row_spec(_TILE_E, 16)]
        + [_fs(a) for a in (*encw, *blkw)],
        out_specs=_row_spec(_TILE_E, LATENT),
        out_shape=jax.ShapeDtypeStruct((EH, LATENT), jnp.float32),
        compiler_params=pltpu.CompilerParams(
            dimension_semantics=("arbitrary",)),
    )(hsrc, ps, pd, *encw, *blkw)


def _edge_call(hsrc, e, blkw):
    grid = (EH // _TILE_E,)
    return pl.pallas_call(
        _edge_body,
        grid=grid,
        in_specs=[_row_spec(_TILE_E, LATENT), _row_spec(_TILE_E, LATENT)]
        + [_fs(a) for a in blkw],
        out_specs=_row_spec(_TILE_E, LATENT),
        out_shape=jax.ShapeDtypeStruct((EH, LATENT), jnp.float32),
        compiler_params=pltpu.CompilerParams(
            dimension_semantics=("arbitrary",)),
    )(hsrc, e, *blkw)


def _node_body(x_in, a0, a1, a2, a3, v1a, v1b, b1, v2, b2, g, be, w1a, w1b,
               x_o, xs_o, xd_o):
    x = x_in[...]
    a = (a0[...] + a1[...]) + (a2[...] + a3[...])
    h = jax.nn.relu(_dot(x, v1a[...]) + _dot(a, v1b[...]) + b1[...])
    xn = x + _ln(_dot(h, v2[...]) + b2[...], g[...], be[...])
    x_o[...] = xn
    xs_o[...] = _dot(xn, w1a[...])
    xd_o[...] = _dot(xn, w1b[...])


def _node_call(x, aggs, nodew, w1a, w1b):
    grid = (N // _TILE_N,)
    out = jax.ShapeDtypeStruct((N, LATENT), jnp.float32)
    return pl.pallas_call(
        _node_body,
        grid=grid,
        in_specs=[_row_spec(_TILE_N, LATENT)] * 5
        + [_fs(a) for a in (*nodew, w1a, w1b)],
        out_specs=[_row_spec(_TILE_N, LATENT)] * 3,
        out_shape=[out, out, out],
        compiler_params=pltpu.CompilerParams(
            dimension_semantics=("arbitrary",)),
    )(x, *aggs, *nodew, w1a, w1b)


def _node_last_body(x_in, a0, a1, a2, a3, v1a, v1b, b1, v2, b2, g, be,
                    d1, db1, d2, db2, wp, pwp, nt, out):
    x = x_in[...]
    a = (a0[...] + a1[...]) + (a2[...] + a3[...])
    h = jax.nn.relu(_dot(x, v1a[...]) + _dot(a, v1b[...]) + b1[...])
    xn = x + _ln(_dot(h, v2[...]) + b2[...], g[...], be[...])
    dh = jax.nn.relu(_dot(xn, d1[...]) + db1[...])
    pred = _dot(dh, d2[...]) + db2[...]
    w = wp[...]
    new = w + (w - pwp[...]) + pred
    out[...] = jnp.where(nt[...] == 0, new, w)


def _node_last_call(x, aggs, nodew, decw, wp, pwp, nt):
    grid = (N // _TILE_N,)
    return pl.pallas_call(
        _node_last_body,
        grid=grid,
        in_specs=[_row_spec(_TILE_N, LATENT)] * 5
        + [_fs(a) for a in (*nodew, *decw)]
        + [_row_spec(_TILE_N, 3), _row_spec(_TILE_N, 3),
           _row_spec(_TILE_N, 1)],
        out_specs=_row_spec(_TILE_N, 3),
        out_shape=jax.ShapeDtypeStruct((N, 3), jnp.float32),
        compiler_params=pltpu.CompilerParams(
            dimension_semantics=("arbitrary",)),
    )(x, *aggs, *nodew, *decw, wp, pwp, nt)


def _wid():
    return lax.axis_index("s") * NC + lax.axis_index("c")


def _sc_pos_gather(table, src3, dst3):
    @functools.partial(
        pl.kernel,
        out_type=(jax.ShapeDtypeStruct((EH, 16), jnp.float32),
                  jax.ShapeDtypeStruct((EH, 16), jnp.float32)),
        mesh=_sc_mesh(),
        scratch_types=(
            pltpu.VMEM((NCHH, CHUNK), jnp.int32),
            pltpu.VMEM((NCHH, CHUNK), jnp.int32),
            pltpu.VMEM((2, CHUNK, 16), jnp.float32),
            pltpu.VMEM((2, CHUNK, 16), jnp.float32),
            (pltpu.SemaphoreType.DMA, pltpu.SemaphoreType.DMA),
            (pltpu.SemaphoreType.DMA, pltpu.SemaphoreType.DMA),
            (pltpu.SemaphoreType.DMA, pltpu.SemaphoreType.DMA),
            (pltpu.SemaphoreType.DMA, pltpu.SemaphoreType.DMA),
        ),
        compiler_params=pltpu.CompilerParams(use_tc_tiling_on_sc=False),
    )
    def k(tab_h, src_h, dst_h, ps_o, pd_o, idxs, idxd, bufs, bufd,
          sga, sgb, soa, sob):
        w = _wid()
        pltpu.sync_copy(src_h.at[w], idxs)
        pltpu.sync_copy(dst_h.at[w], idxd)

        def issue(j, sl):
            return (pltpu.async_copy(tab_h.at[idxs.at[j]], bufs.at[sl],
                                     sga[sl]),
                    pltpu.async_copy(tab_h.at[idxd.at[j]], bufd.at[sl],
                                     sgb[sl]))

        gd = {0: issue(0, 0)}
        od = {}
        for j in range(NCHH):
            sl = j % 2
            if j >= 1:
                for d in od[j - 1]:
                    d.wait()
            if j + 1 < NCHH:
                gd[j + 1] = issue(j + 1, (j + 1) % 2)
            for d in gd[j]:
                d.wait()
            base = w * EPWH + j * CHUNK
            od[j] = (
                pltpu.async_copy(bufs.at[sl], ps_o.at[pl.ds(base, CHUNK)],
                                 soa[sl]),
                pltpu.async_copy(bufd.at[sl], pd_o.at[pl.ds(base, CHUNK)],
                                 sob[sl]),
            )
        for d in od[NCHH - 1]:
            d.wait()

    return k(table, src3, dst3)


def _sc_gather_add(xs, xd, src3, dst3):
    @functools.partial(
        pl.kernel,
        out_type=jax.ShapeDtypeStruct((EH, LATENT), jnp.float32),
        mesh=_sc_mesh(),
        scratch_types=(
            pltpu.VMEM((NCHH, CHUNK), jnp.int32),
            pltpu.VMEM((NCHH, CHUNK), jnp.int32),
            pltpu.VMEM((3, CHUNK, LATENT), jnp.float32),
            pltpu.VMEM((3, CHUNK, LATENT), jnp.float32),
            tuple(pltpu.SemaphoreType.DMA for _ in range(3)),
            tuple(pltpu.SemaphoreType.DMA for _ in range(3)),
            tuple(pltpu.SemaphoreType.DMA for _ in range(3)),
        ),
    )
    def k(xs_h, xd_h, src_h, dst_h, out_h, idxs, idxd, bufa, bufb,
          sga, sgb, so):
        w = _wid()
        pltpu.sync_copy(src_h.at[w], idxs)
        pltpu.sync_copy(dst_h.at[w], idxd)

        def issue(j, sl):
            return (pltpu.async_copy(xs_h.at[idxs.at[j]], bufa.at[sl],
                                     sga[sl]),
                    pltpu.async_copy(xd_h.at[idxd.at[j]], bufb.at[sl],
                                     sgb[sl]))

        gd = {j: issue(j, j) for j in range(min(2, NCHH))}
        od = {}
        for j in range(NCHH):
            sl = j % 3
            if j >= 1:
                od[j - 1].wait()
            if j + 2 < NCHH:
                gd[j + 2] = issue(j + 2, (j + 2) % 3)
            for d in gd[j]:
                d.wait()

            def add_row(r4, carry, _sl=sl):
                for rr in range(4):
                    r = r4 * 4 + rr
                    for cc in range(LATENT // 16):
                        cs = pl.ds(cc * 16, 16)
                        plsc.addupdate(bufa.at[_sl, r, cs], bufb[_sl, r, cs])
                return carry

            lax.fori_loop(0, CHUNK // 4, add_row, 0)
            base = w * EPWH + j * CHUNK
            od[j] = pltpu.async_copy(bufa.at[sl],
                                     out_h.at[pl.ds(base, CHUNK)], so[sl])
        od[NCHH - 1].wait()

    return k(xs, xd, src3, dst3)


def _sc_segment_sum(e, dst3):
    @functools.partial(
        pl.kernel,
        out_type=(jax.ShapeDtypeStruct((N_ACC, LATENT), jnp.float32),
                  jax.ShapeDtypeStruct((N_ACC, LATENT), jnp.float32)),
        mesh=_sc_mesh(),
        scratch_types=(
            pltpu.VMEM((NCHH, CHUNK), jnp.int32),
            pltpu.VMEM((2, CHUNK, LATENT), jnp.float32),
            pltpu.VMEM_SHARED((N_ACC, LATENT), jnp.float32),
            tuple(pltpu.SemaphoreType.DMA for _ in range(2)),
            tuple(pltpu.SemaphoreType.DMA for _ in range(2)),
        ),
    )
    def k(e_h, dst_h, a0_o, a1_o, idx, buf, acc, sld, ssc):
        c = lax.axis_index("c")
        s = lax.axis_index("s")
        w = s * NC + c

        def zero_row(r, carry):
            for cc in range(LATENT // 16):
                buf[0, r, pl.ds(cc * 16, 16)] = jnp.zeros((16,), jnp.float32)
            return carry

        lax.fori_loop(0, CHUNK, zero_row, 0)
        base = s * RPS
        for off in range(0, RPS, CHUNK):
            sz = min(CHUNK, RPS - off)
            pltpu.sync_copy(buf.at[0, pl.ds(0, sz)],
                            acc.at[pl.ds(base + off, sz)])
        plsc.subcore_barrier()

        pltpu.sync_copy(dst_h.at[w], idx)

        def load(j, sl):
            return pltpu.async_copy(
                e_h.at[pl.ds(w * EPWH + j * CHUNK, CHUNK)], buf.at[sl],
                sld[sl])

        ld = {0: load(0, 0)}
        sd = {}
        for j in range(NCHH):
            sl = j % 2
            if j >= 1:
                sd[j - 1].wait()
            if j + 1 < NCHH:
                ld[j + 1] = load(j + 1, (j + 1) % 2)
            ld[j].wait()
            sd[j] = pltpu.async_copy(buf.at[sl], acc.at[idx.at[j]], ssc[sl],
                                     add=True)
        sd[NCHH - 1].wait()
        plsc.subcore_barrier()

        @pl.when(c == 0)
        def _():
            pltpu.sync_copy(acc.at[pl.ds(base, RPS)],
                            a0_o.at[pl.ds(base, RPS)])

        @pl.when(c == 1)
        def _():
            pltpu.sync_copy(acc.at[pl.ds(base, RPS)],
                            a1_o.at[pl.ds(base, RPS)])

    return k(e, dst3)


def _mlp_w(p):
    return (p["W1"], p["b1"].reshape(1, -1), p["W2"], p["b2"].reshape(1, -1),
            p["g"].reshape(1, -1), p["be"].reshape(1, -1))


def kernel(world_pos, prev_world_pos, mesh_pos, node_type, edge_index, params):
    f32 = jnp.float32
    src = edge_index[0].astype(jnp.int32)
    dst = edge_index[1].astype(jnp.int32)
    pad = E_PAD - E
    spread = (jnp.arange(pad, dtype=jnp.int32) * 131) % N
    src4 = jnp.concatenate([src, spread]).reshape(2, NW, NCHH, CHUNK)
    dstg4 = jnp.concatenate([dst, spread]).reshape(2, NW, NCHH, CHUNK)
    dsts4 = jnp.concatenate(
        [dst, N + (jnp.arange(pad, dtype=jnp.int32) % (N_ACC - N))]
    ).reshape(2, NW, NCHH, CHUNK)

    wp = world_pos.astype(f32)
    pwp = prev_world_pos.astype(f32)
    mp = mesh_pos.astype(f32)
    nt = node_type.astype(jnp.int32).reshape(N, 1)
    p16 = jnp.concatenate([wp, mp, jnp.zeros((N, 10), f32)], axis=1)

    pn = params["enc_node"]
    w1p = jnp.concatenate([pn["W1"], jnp.zeros((4, LATENT), f32)], axis=0)
    encn = (w1p, pn["b1"].reshape(1, -1), pn["W2"], pn["b2"].reshape(1, -1),
            pn["g"].reshape(1, -1), pn["be"].reshape(1, -1))
    ence = _mlp_w(params["enc_edge"])

    edge_w = []
    for pe in params["blocks_edge"]:
        w1 = pe["W1"]
        edge_w.append(dict(
            w1a=w1[0:LATENT], w1b=w1[LATENT:2 * LATENT],
            blk=(w1[2 * LATENT:3 * LATENT], pe["b1"].reshape(1, -1),
                 pe["W2"], pe["b2"].reshape(1, -1), pe["g"].reshape(1, -1),
                 pe["be"].reshape(1, -1))))
    node_w = []
    for pnb in params["blocks_node"]:
        v1 = pnb["W1"]
        node_w.append((v1[0:LATENT], v1[LATENT:2 * LATENT],
                       pnb["b1"].reshape(1, -1), pnb["W2"],
                       pnb["b2"].reshape(1, -1), pnb["g"].reshape(1, -1),
                       pnb["be"].reshape(1, -1)))
    d = params["dec"]
    decw = (d["W1"], d["b1"].reshape(1, -1), d["W2"], d["b2"].reshape(1, -1))

    x, xs, xd = _enc_node_call(wp, pwp, nt, *encn,
                               edge_w[0]["w1a"], edge_w[0]["w1b"])
    pos = [_sc_pos_gather(p16, src4[h], dstg4[h]) for h in range(2)]

    eh = [None, None]
    for b in range(N_BLOCKS):
        aggs = []
        for h in range(2):
            hsrc = _sc_gather_add(xs, xd, src4[h], dstg4[h])
            if b == 0:
                eh[h] = _edge0_call(hsrc, pos[h][0], pos[h][1], ence,
                                    edge_w[0]["blk"])
            else:
                eh[h] = _edge_call(hsrc, eh[h], edge_w[b]["blk"])
            aggs.extend(_sc_segment_sum(eh[h], dsts4[h]))
        if b < N_BLOCKS - 1:
            x, xs, xd = _node_call(x, aggs, node_w[b],
                                   edge_w[b + 1]["w1a"], edge_w[b + 1]["w1b"])
        else:
            out = _node_last_call(x, aggs, node_w[b], decw, wp, pwp, nt)
    return out
